# baseline (speedup 1.0000x reference)
import functools

import jax
import jax.numpy as jnp
from jax import lax
from jax.experimental import pallas as pl
from jax.experimental.pallas import tpu as pltpu, tpu_sc as plsc

_NUM_FIELDS = 26
_VOCAB = 100000
_EMBED_DIM = 16
_BATCH = 4096

_info = plsc.get_sparse_core_info()
_NC, _NS, _L = _info.num_cores, _info.num_subcores, _info.num_lanes
_NW = _NC * _NS
_BPW = _BATCH // _NW
_NCHUNK = _BPW // _L
_ROWS_W = _BPW * _NUM_FIELDS
_GCHUNKS = _ROWS_W // _BPW
_BCHUNKS = _ROWS_W // _L


def _body(idx_hbm, emb_hbm, fo_hbm, out_emb, out_fo,
          idx2d, idx_fmaj, idx_bmaj, emb_buf, fo_buf, acc,
          sem_g, sem_fo):
    wid = lax.axis_index("s") * _NC + lax.axis_index("c")
    base = wid * _BPW

    pltpu.sync_copy(idx_hbm.at[:, pl.ds(base, _BPW)], idx2d)

    lane = lax.iota(jnp.int32, 16)

    def build_fmaj(f, carry):
        off = f * _VOCAB
        for c in range(_NCHUNK):
            idx_fmaj[f, pl.ds(c * _L, _L)] = idx2d[f, pl.ds(c * _L, _L)] + off
        return carry

    lax.fori_loop(0, _NUM_FIELDS, build_fmaj, 0)

    def build_bmaj(k, fb):
        f, b = fb
        v = plsc.load_gather(idx_fmaj, [f, b])
        idx_bmaj[pl.ds(k * _L, _L)] = v
        wrap = f >= jnp.int32(_NUM_FIELDS - _L)
        f_n = jnp.where(wrap, f - (_NUM_FIELDS - _L), f + _L)
        b_n = jnp.where(wrap, b + 1, b)
        return (f_n, b_n)

    lax.fori_loop(0, _BCHUNKS, build_bmaj,
                  (lane, jnp.zeros((_L,), jnp.int32)))

    emb_cps = [
        pltpu.async_copy(emb_hbm.at[idx_bmaj.at[pl.ds(c * _BPW, _BPW)]],
                         emb_buf.at[pl.ds(c * _BPW, _BPW)], sem_g)
        for c in range(_GCHUNKS)
    ]

    fo_cps = [
        pltpu.async_copy(fo_hbm.at[idx_fmaj.at[f]],
                         fo_buf.at[pl.ds(f * _BPW, _BPW)], sem_fo)
        for f in range(_NUM_FIELDS)
    ]

    for cp in fo_cps:
        cp.wait()

    def reduce_fo(f, accs):
        return tuple(a + fo_buf[pl.ds(f * _BPW + c * _L, _L)]
                     for c, a in enumerate(accs))

    accs = tuple(jnp.zeros((_L,), jnp.float32) for _ in range(_NCHUNK))
    accs = lax.fori_loop(0, _NUM_FIELDS, reduce_fo, accs)
    for c in range(_NCHUNK):
        acc[pl.ds(c * _L, _L)] = accs[c]
    pltpu.sync_copy(acc, out_fo.at[pl.ds(base, _BPW)])

    for cp in emb_cps:
        cp.wait()
    pltpu.sync_copy(emb_buf, out_emb.at[pl.ds(wid * _ROWS_W, _ROWS_W)])


_fe_kernel = functools.partial(
    pl.kernel,
    out_type=[
        jax.ShapeDtypeStruct((_BATCH * _NUM_FIELDS, _EMBED_DIM), jnp.float32),
        jax.ShapeDtypeStruct((_BATCH,), jnp.float32),
    ],
    mesh=plsc.VectorSubcoreMesh(core_axis_name="c", subcore_axis_name="s"),
    compiler_params=pltpu.CompilerParams(use_tc_tiling_on_sc=False,
                                         needs_layout_passes=False),
    scratch_types=[
        pltpu.VMEM((_NUM_FIELDS, _BPW), jnp.int32),
        pltpu.VMEM((_NUM_FIELDS, _BPW), jnp.int32),
        pltpu.VMEM((_ROWS_W,), jnp.int32),
        pltpu.VMEM((_ROWS_W, _EMBED_DIM), jnp.float32),
        pltpu.VMEM((_NUM_FIELDS * _BPW,), jnp.float32),
        pltpu.VMEM((_BPW,), jnp.float32),
        pltpu.SemaphoreType.DMA,
        pltpu.SemaphoreType.DMA,
    ],
)(_body)


def kernel(indices, emb_tables, fo_tables):
    idx = indices.astype(jnp.int32)
    emb_flat = emb_tables.reshape(_NUM_FIELDS * _VOCAB, _EMBED_DIM)
    fo_flat = fo_tables.reshape(_NUM_FIELDS * _VOCAB)
    out_emb, out_fo = _fe_kernel(idx, emb_flat, fo_flat)
    first_order = out_fo.reshape(_BATCH, 1)
    field_embeddings = out_emb.reshape(_BATCH, _NUM_FIELDS, _EMBED_DIM)
    flat_embeddings = out_emb.reshape(_BATCH, _NUM_FIELDS * _EMBED_DIM)
    return (first_order, field_embeddings, flat_embeddings)


# speedup vs baseline: 7.9749x; 7.9749x over previous
import functools

import jax
import jax.numpy as jnp
from jax import lax
from jax.experimental import pallas as pl
from jax.experimental.pallas import tpu as pltpu, tpu_sc as plsc

_NUM_FIELDS = 26
_VOCAB = 100000
_EMBED_DIM = 16
_BATCH = 4096

_info = plsc.get_sparse_core_info()
_NC, _NS, _L = _info.num_cores, _info.num_subcores, _info.num_lanes
_NW = _NC * _NS
_NTASK = _NUM_FIELDS * _EMBED_DIM
_TPW = _NTASK // _NW
_BCHUNK = _BATCH // _L
_FO_PER_SC = _NUM_FIELDS // _NC


def _sweep(tbl_row_src, idx_hbm, f, tbl_buf, idxf_buf, out_row):
    pltpu.sync_copy(tbl_row_src, tbl_buf)
    pltpu.sync_copy(idx_hbm.at[f, :], idxf_buf)

    def gather(c, carry):
        iv = idxf_buf[pl.ds(c * _L, _L)]
        out_row[pl.ds(c * _L, _L)] = plsc.load_gather(tbl_buf, [iv])
        return carry

    lax.fori_loop(0, _BCHUNK, gather, 0)


def _body(idx_hbm, emb_hbm, fo_hbm, out_emb, out_fl, out_fo,
          tbl_buf, idxf_buf, out_row, fo_shared):
    core = lax.axis_index("c")
    sub = lax.axis_index("s")
    wid = sub * _NC + core

    @pl.when(sub < _FO_PER_SC)
    def _():
        f = core * _FO_PER_SC + sub
        _sweep(fo_hbm.at[f, :], idx_hbm, f, tbl_buf, idxf_buf, out_row)
        pltpu.sync_copy(out_row, fo_shared.at[pl.ds(sub * _BATCH, _BATCH)])

    plsc.subcore_barrier()

    @pl.when(sub == _NS - 1)
    def _():
        for k in range(_FO_PER_SC):
            pltpu.sync_copy(fo_shared.at[pl.ds(k * _BATCH, _BATCH)],
                            tbl_buf.at[pl.ds(k * _BATCH, _BATCH)])

        def red(c, carry):
            acc = tbl_buf[pl.ds(c * _L, _L)]
            for k in range(1, _FO_PER_SC):
                acc = acc + tbl_buf[pl.ds(k * _BATCH + c * _L, _L)]
            out_row[pl.ds(c * _L, _L)] = acc
            return carry

        lax.fori_loop(0, _BCHUNK, red, 0)
        pltpu.sync_copy(out_row, out_fo.at[core, :])

    for j in range(_TPW):
        p = wid * _TPW + j
        f = lax.shift_right_logical(p, 4)
        d = lax.bitwise_and(p, _EMBED_DIM - 1)
        _sweep(emb_hbm.at[f, d, :], idx_hbm, f, tbl_buf, idxf_buf, out_row)
        pltpu.sync_copy(out_row, out_emb.at[f, d, :])
        pltpu.sync_copy(out_row, out_fl.at[p, :])


_fe_kernel = functools.partial(
    pl.kernel,
    out_type=[
        jax.ShapeDtypeStruct((_NUM_FIELDS, _EMBED_DIM, _BATCH), jnp.float32),
        jax.ShapeDtypeStruct((_NUM_FIELDS * _EMBED_DIM, _BATCH), jnp.float32),
        jax.ShapeDtypeStruct((_NC, _BATCH), jnp.float32),
    ],
    mesh=plsc.VectorSubcoreMesh(core_axis_name="c", subcore_axis_name="s"),
    compiler_params=pltpu.CompilerParams(use_tc_tiling_on_sc=True,
                                         needs_layout_passes=False),
    scratch_types=[
        pltpu.VMEM((_VOCAB,), jnp.float32),
        pltpu.VMEM((_BATCH,), jnp.int32),
        pltpu.VMEM((_BATCH,), jnp.float32),
        pltpu.VMEM_SHARED((_FO_PER_SC * _BATCH,), jnp.float32),
    ],
)(_body)


def kernel(indices, emb_tables, fo_tables):
    idx = indices.astype(jnp.int32)
    emb_t = jnp.transpose(emb_tables, (0, 2, 1))
    fo_t = fo_tables.reshape(_NUM_FIELDS, _VOCAB)
    out_emb, out_fl, out_fo = _fe_kernel(idx, emb_t, fo_t)
    first_order = (out_fo[0] + out_fo[1]).reshape(_BATCH, 1)
    field_embeddings = jnp.transpose(out_emb, (2, 0, 1))
    flat_embeddings = jnp.transpose(out_fl, (1, 0))
    return (first_order, field_embeddings, flat_embeddings)


# speedup vs baseline: 9.1064x; 1.1419x over previous
import functools

import jax
import jax.numpy as jnp
from jax import lax
from jax.experimental import pallas as pl
from jax.experimental.pallas import tpu as pltpu, tpu_sc as plsc

_NUM_FIELDS = 26
_VOCAB = 100000
_EMBED_DIM = 16
_BATCH = 4096

_info = plsc.get_sparse_core_info()
_NC, _NS, _L = _info.num_cores, _info.num_subcores, _info.num_lanes
_NW = _NC * _NS
_NTASK = _NUM_FIELDS * _EMBED_DIM
_TPW = _NTASK // _NW
_BCHUNK = _BATCH // _L
_FO_PER_SC = _NUM_FIELDS // _NC
_UNROLL = 8


def _sweep(tbl_row_src, idx_hbm, f, tbl_buf, idxf_buf, out_row):
    pltpu.sync_copy(tbl_row_src, tbl_buf)
    pltpu.sync_copy(idx_hbm.at[f, :], idxf_buf)

    def gather(c, carry):
        for u in range(_UNROLL):
            off = (c * _UNROLL + u) * _L
            iv = idxf_buf[pl.ds(off, _L)]
            out_row[pl.ds(off, _L)] = plsc.load_gather(tbl_buf, [iv])
        return carry

    lax.fori_loop(0, _BCHUNK // _UNROLL, gather, 0)


def _body(idx_hbm, emb_hbm, fo_hbm, out_emb, out_fl, out_fo,
          tbl_buf, idxf_buf, out_row, fo_shared):
    core = lax.axis_index("c")
    sub = lax.axis_index("s")
    wid = sub * _NC + core

    @pl.when(sub < _FO_PER_SC)
    def _():
        f = core * _FO_PER_SC + sub
        _sweep(fo_hbm.at[f, 0, :], idx_hbm, f, tbl_buf, idxf_buf, out_row)
        pltpu.sync_copy(out_row, fo_shared.at[pl.ds(sub * _BATCH, _BATCH)])

    plsc.subcore_barrier()

    @pl.when(sub == _NS - 1)
    def _():
        for k in range(_FO_PER_SC):
            pltpu.sync_copy(fo_shared.at[pl.ds(k * _BATCH, _BATCH)],
                            tbl_buf.at[pl.ds(k * _BATCH, _BATCH)])

        def red(c, carry):
            acc = tbl_buf[pl.ds(c * _L, _L)]
            for k in range(1, _FO_PER_SC):
                acc = acc + tbl_buf[pl.ds(k * _BATCH + c * _L, _L)]
            out_row[pl.ds(c * _L, _L)] = acc
            return carry

        lax.fori_loop(0, _BCHUNK, red, 0)
        pltpu.sync_copy(out_row, out_fo.at[core, :])

    for j in range(_TPW):
        p = wid * _TPW + j
        f = lax.shift_right_logical(p, 4)
        d = lax.bitwise_and(p, _EMBED_DIM - 1)
        _sweep(emb_hbm.at[f, d, :], idx_hbm, f, tbl_buf, idxf_buf, out_row)
        pltpu.sync_copy(out_row, out_emb.at[f, d, :])
        pltpu.sync_copy(out_row, out_fl.at[p, :])


_fe_kernel = functools.partial(
    pl.kernel,
    out_type=[
        jax.ShapeDtypeStruct((_NUM_FIELDS, _EMBED_DIM, _BATCH), jnp.float32),
        jax.ShapeDtypeStruct((_NUM_FIELDS * _EMBED_DIM, _BATCH), jnp.float32),
        jax.ShapeDtypeStruct((_NC, _BATCH), jnp.float32),
    ],
    mesh=plsc.VectorSubcoreMesh(core_axis_name="c", subcore_axis_name="s"),
    compiler_params=pltpu.CompilerParams(use_tc_tiling_on_sc=True,
                                         needs_layout_passes=False),
    scratch_types=[
        pltpu.VMEM((_VOCAB,), jnp.float32),
        pltpu.VMEM((_BATCH,), jnp.int32),
        pltpu.VMEM((_BATCH,), jnp.float32),
        pltpu.VMEM_SHARED((_FO_PER_SC * _BATCH,), jnp.float32),
    ],
)(_body)


def kernel(indices, emb_tables, fo_tables):
    idx = indices.astype(jnp.int32)
    emb_t = jnp.transpose(emb_tables, (0, 2, 1))
    fo_t = jnp.transpose(fo_tables, (0, 2, 1))
    out_emb, out_fl, out_fo = _fe_kernel(idx, emb_t, fo_t)
    first_order = (out_fo[0] + out_fo[1]).reshape(_BATCH, 1)
    field_embeddings = jnp.transpose(out_emb, (2, 0, 1))
    flat_embeddings = jnp.transpose(out_fl, (1, 0))
    return (first_order, field_embeddings, flat_embeddings)


# speedup vs baseline: 9.8251x; 1.0789x over previous
import functools

import jax
import jax.numpy as jnp
from jax import lax
from jax.experimental import pallas as pl
from jax.experimental.pallas import tpu as pltpu, tpu_sc as plsc

_NUM_FIELDS = 26
_VOCAB = 100000
_EMBED_DIM = 16
_BATCH = 4096
_SPLIT = 49920
_H1 = _VOCAB - _SPLIT

_info = plsc.get_sparse_core_info()
_NC, _NS, _L = _info.num_cores, _info.num_subcores, _info.num_lanes
_NW = _NC * _NS
_NTASK = _NUM_FIELDS * _EMBED_DIM
_TPW = _NTASK // _NW
_BCHUNK = _BATCH // _L
_FO_PER_SC = _NUM_FIELDS // _NC
_UNROLL = 8


def _gather_half(half_buf, idx_buf, out_row, h):
    def gather(c, carry):
        for u in range(_UNROLL):
            off = (c * _UNROLL + u) * _L
            iv = idx_buf[pl.ds(off, _L)]
            if h == 0:
                m = iv < _SPLIT
                val = plsc.load_gather(half_buf, [iv], mask=m)
                out_row[pl.ds(off, _L)] = val
            else:
                m = iv >= _SPLIT
                val = plsc.load_gather(half_buf, [iv - _SPLIT], mask=m)
                prev = out_row[pl.ds(off, _L)]
                out_row[pl.ds(off, _L)] = jnp.where(m, val, prev)
        return carry

    lax.fori_loop(0, _BCHUNK // _UNROLL, gather, 0)


def _body(idx_hbm, emb_hbm, fo_hbm, out_emb, out_fl, out_fo,
          h0, h1, idx_a, idx_b, row_a, row_b, fo_shared,
          sem0, sem1, semw, semi):
    core = lax.axis_index("c")
    sub = lax.axis_index("s")
    wid = sub * _NC + core
    halves = (h0, h1)
    hsems = (sem0, sem1)
    idxbufs = (idx_a, idx_b)
    rows = (row_a, row_b)

    @pl.when(sub < _FO_PER_SC)
    def _():
        f = core * _FO_PER_SC + sub
        pltpu.sync_copy(idx_hbm.at[f, :], idx_a)
        pltpu.sync_copy(fo_hbm.at[f, 0, pl.ds(0, _SPLIT)], h0)
        _gather_half(h0, idx_a, row_a, 0)
        pltpu.sync_copy(fo_hbm.at[f, 0, pl.ds(_SPLIT, _H1)], h1)
        _gather_half(h1, idx_a, row_a, 1)
        pltpu.sync_copy(row_a, fo_shared.at[pl.ds(sub * _BATCH, _BATCH)])

    plsc.subcore_barrier()

    @pl.when(sub == _NS - 1)
    def _():
        for k in range(_FO_PER_SC - 1):
            pltpu.sync_copy(fo_shared.at[pl.ds(k * _BATCH, _BATCH)],
                            h0.at[pl.ds(k * _BATCH, _BATCH)])
        pltpu.sync_copy(
            fo_shared.at[pl.ds((_FO_PER_SC - 1) * _BATCH, _BATCH)],
            h1.at[pl.ds(0, _BATCH)])

        def red(c, carry):
            acc = h1[pl.ds(c * _L, _L)]
            for k in range(_FO_PER_SC - 1):
                acc = acc + h0[pl.ds(k * _BATCH + c * _L, _L)]
            row_a[pl.ds(c * _L, _L)] = acc
            return carry

        lax.fori_loop(0, _BCHUNK, red, 0)
        pltpu.sync_copy(row_a, out_fo.at[core, :])

    def fd(j):
        p = wid * _TPW + j
        return (lax.shift_right_logical(p, 4),
                lax.bitwise_and(p, _EMBED_DIM - 1), p)

    units = [(j, h) for j in range(_TPW) for h in range(2)]

    f0, d0, _ = fd(0)
    pltpu.async_copy(idx_hbm.at[f0, :], idx_a, semi).wait()
    stage_cps = {0: pltpu.async_copy(
        emb_hbm.at[f0, d0, pl.ds(0, _SPLIT)], h0, sem0)}

    idx_cps = {}
    wr_cps = {}

    for u, (j, h) in enumerate(units):
        f, d, p = fd(j)
        idx_cur = idxbufs[j % 2]
        out_row = rows[j % 2]

        stage_cps[u].wait()

        if u + 1 < len(units):
            jn, hn = units[u + 1]
            fn, dn, _ = fd(jn)
            stage_cps[u + 1] = pltpu.async_copy(
                emb_hbm.at[fn, dn, pl.ds(0, _SPLIT)] if hn == 0 else emb_hbm.at[fn, dn, pl.ds(_SPLIT, _H1)],
                halves[(u + 1) % 2], hsems[(u + 1) % 2])

        if h == 0:
            if j - 2 in wr_cps:
                for cp in wr_cps.pop(j - 2):
                    cp.wait()
            if j + 1 < _TPW:
                fn1, _, _ = fd(j + 1)
                idx_cps[j + 1] = pltpu.async_copy(
                    idx_hbm.at[fn1, :], idxbufs[(j + 1) % 2], semi)
            if j in idx_cps:
                idx_cps.pop(j).wait()

        _gather_half(halves[u % 2], idx_cur, out_row, h)

        if h == 1:
            wr_cps[j] = [
                pltpu.async_copy(out_row, out_emb.at[f, d, :], semw),
                pltpu.async_copy(out_row, out_fl.at[p, :], semw),
            ]

    for cps in wr_cps.values():
        for cp in cps:
            cp.wait()


_fe_kernel = functools.partial(
    pl.kernel,
    out_type=[
        jax.ShapeDtypeStruct((_NUM_FIELDS, _EMBED_DIM, _BATCH), jnp.float32),
        jax.ShapeDtypeStruct((_NUM_FIELDS * _EMBED_DIM, _BATCH), jnp.float32),
        jax.ShapeDtypeStruct((_NC, _BATCH), jnp.float32),
    ],
    mesh=plsc.VectorSubcoreMesh(core_axis_name="c", subcore_axis_name="s"),
    compiler_params=pltpu.CompilerParams(use_tc_tiling_on_sc=True,
                                         needs_layout_passes=False),
    scratch_types=[
        pltpu.VMEM((_SPLIT,), jnp.float32),
        pltpu.VMEM((_H1,), jnp.float32),
        pltpu.VMEM((_BATCH,), jnp.int32),
        pltpu.VMEM((_BATCH,), jnp.int32),
        pltpu.VMEM((_BATCH,), jnp.float32),
        pltpu.VMEM((_BATCH,), jnp.float32),
        pltpu.VMEM_SHARED((_FO_PER_SC * _BATCH,), jnp.float32),
        pltpu.SemaphoreType.DMA,
        pltpu.SemaphoreType.DMA,
        pltpu.SemaphoreType.DMA,
        pltpu.SemaphoreType.DMA,
    ],
)(_body)


def kernel(indices, emb_tables, fo_tables):
    idx = indices.astype(jnp.int32)
    emb_t = jnp.transpose(emb_tables, (0, 2, 1))
    fo_t = jnp.transpose(fo_tables, (0, 2, 1))
    out_emb, out_fl, out_fo = _fe_kernel(idx, emb_t, fo_t)
    first_order = (out_fo[0] + out_fo[1]).reshape(_BATCH, 1)
    field_embeddings = jnp.transpose(out_emb, (2, 0, 1))
    flat_embeddings = jnp.transpose(out_fl, (1, 0))
    return (first_order, field_embeddings, flat_embeddings)
